# Initial kernel scaffold; baseline (speedup 1.0000x reference)
import jax
import jax.numpy as jnp
from jax.experimental import pallas as pl


def kernel(input, adj, h0, lamda, l, weight, alpha):
    raise NotImplementedError("write your pallas kernel here")


# speedup vs baseline: 1.0308x; 1.0308x over previous
import jax
import jax.numpy as jnp
from jax.experimental import pallas as pl
from jax.experimental.pallas import tpu as pltpu

_N = 10000
_D = 128
_BM = 400


def _gcn_kernel(adj_ref, x_ref, h0_ref, w_ref, alpha_ref, out_ref):
    hi = jnp.dot(adj_ref[...], x_ref[...], preferred_element_type=jnp.float32)
    a = jax.nn.sigmoid(alpha_ref[...]) * 0.5
    support = (1.0 - a) * hi + a * h0_ref[...]
    out_ref[...] = 0.25 * jnp.dot(
        support, w_ref[...], preferred_element_type=jnp.float32
    ) + 0.75 * support


def kernel(input, adj, h0, lamda, l, weight, alpha):
    del lamda, l
    n, d_in = input.shape
    alpha2d = alpha.reshape(1, 1)
    return pl.pallas_call(
        _gcn_kernel,
        grid=(n // _BM,),
        in_specs=[
            pl.BlockSpec((_BM, n), lambda i: (i, 0)),
            pl.BlockSpec((n, d_in), lambda i: (0, 0)),
            pl.BlockSpec((_BM, d_in), lambda i: (i, 0)),
            pl.BlockSpec(weight.shape, lambda i: (0, 0)),
            pl.BlockSpec((1, 1), lambda i: (0, 0)),
        ],
        out_specs=pl.BlockSpec((_BM, d_in), lambda i: (i, 0)),
        out_shape=jax.ShapeDtypeStruct((n, weight.shape[1]), jnp.float32),
    )(adj, input, h0, weight, alpha2d)
